# baseline (speedup 1.0000x reference)
import functools

import jax
import jax.numpy as jnp
from jax import lax
from jax.experimental import pallas as pl
from jax.experimental.pallas import tpu as pltpu
from jax.experimental.pallas import tpu_sc as plsc

K_TOP_N = 32
NEG_INF = float("-inf")


def _score_topk_body(q_ref, t_ref, sco_ref, ids_ref, mx_s, arg_s):
    i = pl.program_id(0)
    n_steps = pl.num_programs(0)
    n_q = q_ref.shape[0]

    s = lax.dot_general(
        t_ref[...], q_ref[...],
        dimension_numbers=(((1,), (1,)), ((), ())),
        preferred_element_type=jnp.float32,
    )
    rows_per_block = mx_s.shape[0] // n_steps
    neg_iota = (-lax.broadcasted_iota(jnp.int32, (128, n_q), 0)
                ).astype(jnp.float32)
    for r in range(rows_per_block):
        sub = lax.slice_in_dim(s, r * 128, (r + 1) * 128, axis=0)
        m = jnp.max(sub, axis=0, keepdims=True)
        na = jnp.max(jnp.where(sub == m, neg_iota, NEG_INF),
                     axis=0, keepdims=True)
        row = i * rows_per_block + r
        mx_s[pl.ds(row, 1), :] = m
        arg_s[pl.ds(row, 1), :] = (-na).astype(jnp.int32)

    @pl.when(i == n_steps - 1)
    def _topk():
        n_rows = mx_s.shape[0]
        qc = 256
        for j in range(n_q // qc):
            sj = mx_s[:, pl.ds(j * qc, qc)]
            row_iota = lax.broadcasted_iota(jnp.int32, (n_rows, qc), 0)
            neg_gid = -(row_iota * 128 + arg_s[:, pl.ds(j * qc, qc)]
                        ).astype(jnp.float32)
            for k in range(K_TOP_N):
                m = jnp.max(sj, axis=0, keepdims=True)
                eq = sj == m
                ng = jnp.max(jnp.where(eq, neg_gid, NEG_INF),
                             axis=0, keepdims=True)
                sco_ref[pl.ds(k, 1), pl.ds(j * qc, qc)] = m
                ids_ref[pl.ds(k, 1), pl.ds(j * qc, qc)] = (
                    -ng).astype(jnp.int32)
                sj = jnp.where(eq, NEG_INF, sj)


def _run_score_topk(queries, table2d, rows_per_block=64):
    n_q, d = queries.shape
    n_vec = table2d.shape[0]
    n_rows = n_vec // 128
    vb = rows_per_block * 128
    grid = (n_vec // vb,)
    return pl.pallas_call(
        _score_topk_body,
        grid=grid,
        in_specs=[
            pl.BlockSpec((n_q, d), lambda i: (0, 0)),
            pl.BlockSpec((vb, d), lambda i: (i, 0)),
        ],
        out_specs=[
            pl.BlockSpec((K_TOP_N, n_q), lambda i: (0, 0)),
            pl.BlockSpec((K_TOP_N, n_q), lambda i: (0, 0)),
        ],
        out_shape=[
            jax.ShapeDtypeStruct((K_TOP_N, n_q), jnp.float32),
            jax.ShapeDtypeStruct((K_TOP_N, n_q), jnp.int32),
        ],
        scratch_shapes=[
            pltpu.VMEM((n_rows, n_q), jnp.float32),
            pltpu.VMEM((n_rows, n_q), jnp.int32),
        ],
    )(queries, table2d)


def _run_gather(table2d, ids_3d):
    n_w, n_ch, ch = ids_3d.shape
    d = table2d.shape[1]
    b = n_w * n_ch * ch
    bpw = n_ch * ch
    info = plsc.get_sparse_core_info()
    n_cores = info.num_cores
    mesh = plsc.VectorSubcoreMesh(core_axis_name="c", subcore_axis_name="s")

    @functools.partial(
        pl.kernel,
        mesh=mesh,
        out_type=jax.ShapeDtypeStruct((b, d), jnp.float32),
        scratch_types=[
            pltpu.VMEM((n_ch, ch), jnp.int32),
            pltpu.VMEM((ch, d), jnp.float32),
            pltpu.VMEM((ch, d), jnp.float32),
            pltpu.SemaphoreType.DMA,
            pltpu.SemaphoreType.DMA,
            pltpu.SemaphoreType.DMA,
            pltpu.SemaphoreType.DMA,
        ],
    )
    def gather_kernel(tab_hbm, idx_hbm, out_hbm, idx_v, rows0, rows1,
                      gsem0, gsem1, osem0, osem1):
        wid = lax.axis_index("s") * n_cores + lax.axis_index("c")
        base = wid * bpw
        rows = (rows0, rows1)
        gsem = (gsem0, gsem1)
        osem = (osem0, osem1)
        pltpu.sync_copy(idx_hbm.at[wid], idx_v)
        gath = [None, None]
        outh = [None, None]
        gath[0] = pltpu.async_copy(tab_hbm.at[idx_v.at[0]], rows0, gsem0)
        for c in range(n_ch):
            bf = c & 1
            nb = 1 - bf
            gath[bf].wait()
            outh[bf] = pltpu.async_copy(
                rows[bf], out_hbm.at[pl.ds(base + c * ch, ch)], osem[bf])
            if c + 1 < n_ch:
                if outh[nb] is not None:
                    outh[nb].wait()
                gath[nb] = pltpu.async_copy(
                    tab_hbm.at[idx_v.at[c + 1]], rows[nb], gsem[nb])
        for h in outh:
            if h is not None:
                h.wait()

    return gather_kernel(table2d, ids_3d)


def kernel(queries, table):
    n_q, d = queries.shape
    n_rows, spr, _ = table.shape
    table2d = table.reshape(n_rows * spr, d)

    sco_t, ids_t = _run_score_topk(queries, table2d)
    topk_scores = sco_t.T
    topk_ids = ids_t.T

    flat_ids = topk_ids.reshape(32, -1, 128)
    vals = _run_gather(table2d, flat_ids)
    topk_values = vals.reshape(n_q, K_TOP_N, d)
    return (topk_values, topk_scores, topk_ids)


# speedup vs baseline: 1.1533x; 1.1533x over previous
import functools

import jax
import jax.numpy as jnp
from jax import lax
from jax.experimental import pallas as pl
from jax.experimental.pallas import tpu as pltpu
from jax.experimental.pallas import tpu_sc as plsc

K_TOP_N = 32
NEG_INF = float("-inf")


def _score_topk_body(q_ref, t_ref, sco_ref, ids_ref, mx_s, arg_s):
    i = pl.program_id(0)
    n_steps = pl.num_programs(0)
    n_q = q_ref.shape[0]

    s = lax.dot_general(
        t_ref[...], q_ref[...],
        dimension_numbers=(((1,), (1,)), ((), ())),
        preferred_element_type=jnp.float32,
    )
    rows_per_block = mx_s.shape[0] // n_steps
    neg_iota = (-lax.broadcasted_iota(jnp.int32, (128, n_q), 0)
                ).astype(jnp.float32)
    for r in range(rows_per_block):
        sub = lax.slice_in_dim(s, r * 128, (r + 1) * 128, axis=0)
        m = jnp.max(sub, axis=0, keepdims=True)
        na = jnp.max(jnp.where(sub == m, neg_iota, NEG_INF),
                     axis=0, keepdims=True)
        row = i * rows_per_block + r
        mx_s[pl.ds(row, 1), :] = m
        arg_s[pl.ds(row, 1), :] = (-na).astype(jnp.int32)

    @pl.when(i == n_steps - 1)
    def _topk():
        n_rows = mx_s.shape[0]
        qc = 256
        for j in range(n_q // qc):
            sj = mx_s[:, pl.ds(j * qc, qc)]
            row_iota = lax.broadcasted_iota(jnp.int32, (n_rows, qc), 0)
            neg_gid = -(row_iota * 128 + arg_s[:, pl.ds(j * qc, qc)]
                        ).astype(jnp.float32)
            for k in range(K_TOP_N):
                m = jnp.max(sj, axis=0, keepdims=True)
                eq = sj == m
                ng = jnp.max(jnp.where(eq, neg_gid, NEG_INF),
                             axis=0, keepdims=True)
                sco_ref[pl.ds(k, 1), pl.ds(j * qc, qc)] = m
                ids_ref[pl.ds(k, 1), pl.ds(j * qc, qc)] = (
                    -ng).astype(jnp.int32)
                sj = jnp.where(eq, NEG_INF, sj)


def _run_score_topk(queries, table2d, rows_per_block=32):
    n_q, d = queries.shape
    n_vec = table2d.shape[0]
    n_rows = n_vec // 128
    vb = rows_per_block * 128
    grid = (n_vec // vb,)
    return pl.pallas_call(
        _score_topk_body,
        grid=grid,
        in_specs=[
            pl.BlockSpec((n_q, d), lambda i: (0, 0)),
            pl.BlockSpec((vb, d), lambda i: (i, 0)),
        ],
        out_specs=[
            pl.BlockSpec((K_TOP_N, n_q), lambda i: (0, 0)),
            pl.BlockSpec((K_TOP_N, n_q), lambda i: (0, 0)),
        ],
        out_shape=[
            jax.ShapeDtypeStruct((K_TOP_N, n_q), jnp.float32),
            jax.ShapeDtypeStruct((K_TOP_N, n_q), jnp.int32),
        ],
        scratch_shapes=[
            pltpu.VMEM((n_rows, n_q), jnp.float32),
            pltpu.VMEM((n_rows, n_q), jnp.int32),
        ],
    )(queries, table2d)


def _run_gather(table2d, ids_3d):
    n_w, n_ch, ch = ids_3d.shape
    d = table2d.shape[1]
    b = n_w * n_ch * ch
    bpw = n_ch * ch
    info = plsc.get_sparse_core_info()
    n_cores = info.num_cores
    mesh = plsc.VectorSubcoreMesh(core_axis_name="c", subcore_axis_name="s")

    @functools.partial(
        pl.kernel,
        mesh=mesh,
        out_type=jax.ShapeDtypeStruct((b, d), jnp.float32),
        scratch_types=[
            pltpu.VMEM((n_ch, ch), jnp.int32),
            pltpu.VMEM((ch, d), jnp.float32),
            pltpu.VMEM((ch, d), jnp.float32),
            pltpu.SemaphoreType.DMA,
            pltpu.SemaphoreType.DMA,
            pltpu.SemaphoreType.DMA,
            pltpu.SemaphoreType.DMA,
        ],
    )
    def gather_kernel(tab_hbm, idx_hbm, out_hbm, idx_v, rows0, rows1,
                      gsem0, gsem1, osem0, osem1):
        wid = lax.axis_index("s") * n_cores + lax.axis_index("c")
        base = wid * bpw
        rows = (rows0, rows1)
        gsem = (gsem0, gsem1)
        osem = (osem0, osem1)
        pltpu.sync_copy(idx_hbm.at[wid], idx_v)
        gath = [None, None]
        outh = [None, None]
        gath[0] = pltpu.async_copy(tab_hbm.at[idx_v.at[0]], rows0, gsem0)
        for c in range(n_ch):
            bf = c & 1
            nb = 1 - bf
            gath[bf].wait()
            outh[bf] = pltpu.async_copy(
                rows[bf], out_hbm.at[pl.ds(base + c * ch, ch)], osem[bf])
            if c + 1 < n_ch:
                if outh[nb] is not None:
                    outh[nb].wait()
                gath[nb] = pltpu.async_copy(
                    tab_hbm.at[idx_v.at[c + 1]], rows[nb], gsem[nb])
        for h in outh:
            if h is not None:
                h.wait()

    return gather_kernel(table2d, ids_3d)


def kernel(queries, table):
    n_q, d = queries.shape
    n_rows, spr, _ = table.shape
    table2d = table.reshape(n_rows * spr, d)

    sco_t, ids_t = _run_score_topk(queries, table2d)
    topk_scores = sco_t.T
    topk_ids = ids_t.T

    flat_ids = topk_ids.reshape(32, -1, 128)
    vals = _run_gather(table2d, flat_ids)
    topk_values = vals.reshape(n_q, K_TOP_N, d)
    return (topk_values, topk_scores, topk_ids)
